# baseline (speedup 1.0000x reference)
import functools

import jax
import jax.numpy as jnp
from jax import lax
from jax.experimental import pallas as pl
from jax.experimental.pallas import tpu as pltpu
from jax.experimental.pallas import tpu_sc as plsc

VOCAB = 1000
NEMBED = 32
B, T = 1024, 50
TP = 56
NC, NS = 2, 16
NW = NC * NS
BPW = B // NW


def _dense_body(tok_ref, pos_ref, w_ref, b_ref, g_ref, p_ref):
    w = w_ref[...]
    g_ref[...] = jnp.dot(tok_ref[...], w,
                         preferred_element_type=jnp.float32) + b_ref[...]
    p_ref[...] = jnp.dot(pos_ref[...], w, preferred_element_type=jnp.float32)


def _dense(token_table, pos, W, b2):
    return pl.pallas_call(
        _dense_body,
        out_shape=(
            jax.ShapeDtypeStruct((VOCAB, VOCAB), jnp.float32),
            jax.ShapeDtypeStruct((T, VOCAB), jnp.float32),
        ),
    )(token_table, pos, W, b2)


def _sc_body(g_hbm, p_hbm, idx_hbm, out_hbm, idxb, shP, bufA, sem):
    sid = lax.axis_index("s")
    wid = sid * NC + lax.axis_index("c")
    b0 = wid * BPW
    pltpu.sync_copy(idx_hbm.at[pl.ds(b0, BPW)], idxb)

    @pl.when(sid == 0)
    def _():
        pltpu.sync_copy(p_hbm, shP)

    plsc.subcore_barrier()

    def body(j, carry):
        pltpu.sync_copy(shP, bufA.at[pl.ds(0, T)])
        pltpu.async_copy(g_hbm.at[idxb.at[j]], bufA, sem, add=True).wait()
        pltpu.sync_copy(bufA.at[pl.ds(0, T)],
                        out_hbm.at[pl.ds((b0 + j) * T, T)])
        return carry

    lax.fori_loop(0, BPW, body, 0)


_sc = functools.partial(
    pl.kernel,
    out_type=jax.ShapeDtypeStruct((B * T, VOCAB), jnp.float32),
    mesh=plsc.VectorSubcoreMesh(core_axis_name="c", subcore_axis_name="s"),
    compiler_params=pltpu.CompilerParams(use_tc_tiling_on_sc=False),
    scratch_types=[
        pltpu.VMEM((BPW, TP), jnp.int32),
        pltpu.VMEM_SHARED((T, VOCAB), jnp.float32),
        pltpu.VMEM((TP, VOCAB), jnp.float32),
        pltpu.SemaphoreType.DMA,
    ],
)(_sc_body)


def kernel(idx, token_table, pos_table, W, b):
    g, p = _dense(token_table, pos_table[:T], W, b.reshape(1, VOCAB))
    idxp = jnp.pad(idx, ((0, 0), (0, TP - T)))
    out = _sc(g, p, idxp)
    return out.reshape(B, T, VOCAB)
